# baseline (speedup 1.0000x reference)
import jax
import jax.numpy as jnp
from jax import lax
from jax.experimental import pallas as pl

N_VOCAB = 2048
SEQ_BLK = 512
SEQ_OUT = 8193
N_BLKS = 17


def _onehot_block(ids_ref, out_ref):
    ids = ids_ref[0, 0, 0, :]
    cls = lax.broadcasted_iota(jnp.int32, (SEQ_BLK, N_VOCAB), 1)
    out_ref[0] = (ids[:, None] == cls).astype(jnp.float32)


def kernel(input_ids):
    batch, seq = input_ids.shape
    padded = jnp.pad(
        input_ids.astype(jnp.int32),
        ((0, 0), (1, N_BLKS * SEQ_BLK - seq - 1)),
        constant_values=0,
    )
    ids4 = padded.reshape(batch, N_BLKS, 1, SEQ_BLK)

    return pl.pallas_call(
        _onehot_block,
        grid=(batch, N_BLKS),
        in_specs=[
            pl.BlockSpec((1, 1, 1, SEQ_BLK), lambda b, j: (b, j, 0, 0)),
        ],
        out_specs=pl.BlockSpec((1, SEQ_BLK, N_VOCAB), lambda b, j: (b, j, 0)),
        out_shape=jax.ShapeDtypeStruct((batch, SEQ_OUT, N_VOCAB), jnp.float32),
    )(ids4)
